# baseline (speedup 1.0000x reference)
import functools

import jax
import jax.numpy as jnp
from jax import lax
from jax.experimental import pallas as pl
from jax.experimental.pallas import tpu as pltpu
from jax.experimental.pallas import tpu_sc as plsc

B, C, H, W = 8, 384, 56, 56
HW = H * W
P1 = C // 2
ROWS = B * C
G = 16
NG = P1 // G


def _sc_body(x0, x1, bn1, bn2, thrh, zrow, o0, o1,
             bn1_v, bn2_v, thr_v, sel_v, buf, sem_in, sem_out, sem_s):
    wid = lax.axis_index("s") * 2 + lax.axis_index("c")
    b = (wid % 16) // 2
    half = wid % 2
    is0 = wid < 16

    pltpu.sync_copy(bn1, bn1_v)
    pltpu.sync_copy(bn2, bn2_v)
    pltpu.sync_copy(thrh, thr_v)
    thr = thr_v[...]

    zero_i = jnp.zeros((16,), jnp.int32)
    one_i = jnp.ones((16,), jnp.int32)
    two_i = one_i + one_i
    for k in range(P1 // 16):
        a1 = jnp.abs(bn1_v[pl.ds(P1 + k * 16, 16)])
        a2 = jnp.abs(bn2_v[pl.ds(P1 + k * 16, 16)])
        sel_v[0, pl.ds(k * 16, 16)] = jnp.where(
            a1 > thr, zero_i, jnp.where(a1 < thr, one_i, two_i))
        sel_v[1, pl.ds(k * 16, 16)] = jnp.where(
            a2 > thr, zero_i, jnp.where(a2 < thr, one_i, two_i))

    def copy_half(o, src):
        def grp(g, carry):
            r0 = b * C + g * G
            pltpu.async_copy(src.at[pl.ds(r0, G), :], buf, sem_in).wait()
            pltpu.async_copy(buf, o.at[pl.ds(r0, G), :], sem_out).wait()
            return carry
        lax.fori_loop(0, NG, grp, 0)

    def select_half(o, same, other, srow):
        def grp(g, carry):
            r0 = b * C + P1 + g * G
            sv = sel_v[srow, pl.ds(g * G, 16)]
            for j in range(G):
                s = sv[j]
                @pl.when(s == 0)
                def _():
                    pltpu.async_copy(same.at[r0 + j], buf.at[j], sem_in)
                @pl.when(s == 1)
                def _():
                    pltpu.async_copy(other.at[r0 + j], buf.at[j], sem_in)
                @pl.when(s == 2)
                def _():
                    pltpu.async_copy(zrow, buf.at[j], sem_in)
            for j in range(G):
                pltpu.make_async_copy(zrow, buf.at[j], sem_in).wait()
            pltpu.async_copy(buf, o.at[pl.ds(r0, G), :], sem_out).wait()
            return carry
        lax.fori_loop(0, NG, grp, 0)

    @pl.when(jnp.logical_and(is0, half == 0))
    def _():
        copy_half(o0, x0)

    @pl.when(jnp.logical_and(is0, half == 1))
    def _():
        select_half(o0, x0, x1, 0)

    @pl.when(jnp.logical_and(jnp.logical_not(is0), half == 0))
    def _():
        copy_half(o1, x1)

    @pl.when(jnp.logical_and(jnp.logical_not(is0), half == 1))
    def _():
        select_half(o1, x1, x0, 1)


@jax.jit
def _run(x0, x1, bn1, bn2, thr):
    x0r = x0.reshape(ROWS, HW)
    x1r = x1.reshape(ROWS, HW)
    thrh = jnp.full((16,), thr, jnp.float32)
    zrow = jnp.zeros((HW,), jnp.float32)
    mesh = plsc.VectorSubcoreMesh(core_axis_name="c", subcore_axis_name="s")
    f = pl.kernel(
        _sc_body,
        out_type=[
            jax.ShapeDtypeStruct((ROWS, HW), jnp.float32),
            jax.ShapeDtypeStruct((ROWS, HW), jnp.float32),
        ],
        mesh=mesh,
        scratch_types=[
            pltpu.VMEM((C,), jnp.float32),
            pltpu.VMEM((C,), jnp.float32),
            pltpu.VMEM((16,), jnp.float32),
            pltpu.VMEM((2, P1), jnp.int32),
            pltpu.VMEM((G, HW), jnp.float32),
            pltpu.SemaphoreType.DMA,
            pltpu.SemaphoreType.DMA,
            pltpu.SemaphoreType.DMA,
        ],
    )
    o0, o1 = f(x0r, x1r, bn1, bn2, thrh, zrow)
    return o0.reshape(B, C, H, W), o1.reshape(B, C, H, W)


def kernel(x0, x1, bn1_weight, bn2_weight, bn_threshold):
    return _run(x0, x1, bn1_weight, bn2_weight, bn_threshold)


# speedup vs baseline: 1.0078x; 1.0078x over previous
import jax
import jax.numpy as jnp
from jax import lax
from jax.experimental import pallas as pl
from jax.experimental.pallas import tpu as pltpu
from jax.experimental.pallas import tpu_sc as plsc

B, C, H, W = 8, 384, 56, 56
HW = H * W
P1 = C // 2
ROWS = B * C
G = 16
NG = P1 // G
NP = NG // 2


def _sc_body(x0, x1, bn1, bn2, thrh, zrow, o0, o1,
             bn1_v, bn2_v, thr_v, sel_v, buf0, buf1, sem_in, sem_out):
    wid = lax.axis_index("s") * 2 + lax.axis_index("c")
    b = (wid % 16) // 2
    half = wid % 2
    is0 = wid < 16

    pltpu.sync_copy(bn1, bn1_v)
    pltpu.sync_copy(bn2, bn2_v)
    pltpu.sync_copy(thrh, thr_v)
    thr = thr_v[...]

    zero_i = jnp.zeros((16,), jnp.int32)
    one_i = jnp.ones((16,), jnp.int32)
    two_i = one_i + one_i
    for k in range(P1 // 16):
        a1 = jnp.abs(bn1_v[pl.ds(P1 + k * 16, 16)])
        a2 = jnp.abs(bn2_v[pl.ds(P1 + k * 16, 16)])
        sel_v[0, pl.ds(k * 16, 16)] = jnp.where(
            a1 > thr, zero_i, jnp.where(a1 < thr, one_i, two_i))
        sel_v[1, pl.ds(k * 16, 16)] = jnp.where(
            a2 > thr, zero_i, jnp.where(a2 < thr, one_i, two_i))

    def drain_out(o):
        pltpu.make_async_copy(buf0, o.at[pl.ds(0, G), :], sem_out).wait()

    def wait_in(same):
        pltpu.make_async_copy(same.at[pl.ds(0, G), :], buf0, sem_in).wait()

    def run_half(o, same, other, srow, row0):
        def fire_gather(g, buf):
            r0 = b * C + row0 + g * G
            if srow is None:
                pltpu.async_copy(same.at[pl.ds(r0, G), :], buf, sem_in)
            else:
                sv = sel_v[srow, pl.ds(g * G, 16)]
                for j in range(G):
                    s = sv[j]
                    @pl.when(s == 0)
                    def _():
                        pltpu.async_copy(same.at[r0 + j], buf.at[j], sem_in)
                    @pl.when(s == 1)
                    def _():
                        pltpu.async_copy(other.at[r0 + j], buf.at[j], sem_in)
                    @pl.when(s == 2)
                    def _():
                        pltpu.async_copy(zrow, buf.at[j], sem_in)

        def fire_scatter(g, buf):
            r0 = b * C + row0 + g * G
            pltpu.async_copy(buf, o.at[pl.ds(r0, G), :], sem_out)

        def body(p, carry):
            ga = 2 * p
            gb = 2 * p + 1
            fire_gather(ga, buf0)
            @pl.when(p > 0)
            def _():
                drain_out(o)
            wait_in(same)
            fire_scatter(ga, buf0)
            fire_gather(gb, buf1)
            drain_out(o)
            wait_in(same)
            fire_scatter(gb, buf1)
            return carry

        lax.fori_loop(0, NP, body, 0)
        drain_out(o)

    @pl.when(jnp.logical_and(is0, half == 0))
    def _():
        run_half(o0, x0, x1, None, 0)

    @pl.when(jnp.logical_and(is0, half == 1))
    def _():
        run_half(o0, x0, x1, 0, P1)

    @pl.when(jnp.logical_and(jnp.logical_not(is0), half == 0))
    def _():
        run_half(o1, x1, x0, None, 0)

    @pl.when(jnp.logical_and(jnp.logical_not(is0), half == 1))
    def _():
        run_half(o1, x1, x0, 1, P1)


@jax.jit
def _run(x0, x1, bn1, bn2, thr):
    x0r = x0.reshape(ROWS, HW)
    x1r = x1.reshape(ROWS, HW)
    thrh = jnp.full((16,), thr, jnp.float32)
    zrow = jnp.zeros((HW,), jnp.float32)
    mesh = plsc.VectorSubcoreMesh(core_axis_name="c", subcore_axis_name="s")
    f = pl.kernel(
        _sc_body,
        out_type=[
            jax.ShapeDtypeStruct((ROWS, HW), jnp.float32),
            jax.ShapeDtypeStruct((ROWS, HW), jnp.float32),
        ],
        mesh=mesh,
        scratch_types=[
            pltpu.VMEM((C,), jnp.float32),
            pltpu.VMEM((C,), jnp.float32),
            pltpu.VMEM((16,), jnp.float32),
            pltpu.VMEM((2, P1), jnp.int32),
            pltpu.VMEM((G, HW), jnp.float32),
            pltpu.VMEM((G, HW), jnp.float32),
            pltpu.SemaphoreType.DMA,
            pltpu.SemaphoreType.DMA,
        ],
    )
    o0, o1 = f(x0r, x1r, bn1, bn2, thrh, zrow)
    return o0.reshape(B, C, H, W), o1.reshape(B, C, H, W)


def kernel(x0, x1, bn1_weight, bn2_weight, bn_threshold):
    return _run(x0, x1, bn1_weight, bn2_weight, bn_threshold)


# speedup vs baseline: 7.1626x; 7.1074x over previous
import jax
import jax.numpy as jnp
from jax.experimental import pallas as pl
from jax.experimental.pallas import tpu as pltpu

B, C, H, W = 8, 384, 56, 56
P1 = C // 2
N = B * H * W
RB = 512
GRID = N // RB


def _body(thr_ref, bn1_ref, bn2_ref, x0_ref, x1_ref, o0_ref, o1_ref):
    thr = thr_ref[0, 0]
    c_idx = jax.lax.broadcasted_iota(jnp.int32, (1, C), 1)
    first = c_idx < P1
    bn1 = jnp.abs(bn1_ref[...])
    bn2 = jnp.abs(bn2_ref[...])
    keep0 = jnp.logical_or(first, bn1 > thr)
    take0 = jnp.logical_and(jnp.logical_not(first), bn1 < thr)
    keep1 = jnp.logical_or(first, bn2 > thr)
    take1 = jnp.logical_and(jnp.logical_not(first), bn2 < thr)
    x0 = x0_ref[...]
    x1 = x1_ref[...]
    zero = jnp.zeros_like(x0)
    o0_ref[...] = jnp.where(keep0, x0, jnp.where(take0, x1, zero))
    o1_ref[...] = jnp.where(keep1, x1, jnp.where(take1, x0, zero))


@jax.jit
def _run(x0, x1, bn1, bn2, thr):
    x0r = x0.transpose(0, 2, 3, 1).reshape(N, C)
    x1r = x1.transpose(0, 2, 3, 1).reshape(N, C)
    bn1r = bn1.reshape(1, C)
    bn2r = bn2.reshape(1, C)
    thr_arr = jnp.asarray(thr, jnp.float32).reshape(1, 1)
    data_spec = pl.BlockSpec((RB, C), lambda i: (i, 0))
    vec_spec = pl.BlockSpec((1, C), lambda i: (0, 0))
    thr_spec = pl.BlockSpec((1, 1), lambda i: (0, 0))
    o0, o1 = pl.pallas_call(
        _body,
        grid=(GRID,),
        in_specs=[thr_spec, vec_spec, vec_spec, data_spec, data_spec],
        out_specs=[data_spec, data_spec],
        out_shape=[
            jax.ShapeDtypeStruct((N, C), jnp.float32),
            jax.ShapeDtypeStruct((N, C), jnp.float32),
        ],
        compiler_params=pltpu.CompilerParams(
            dimension_semantics=("parallel",),
        ),
    )(thr_arr, bn1r, bn2r, x0r, x1r)
    o0 = o0.reshape(B, H, W, C).transpose(0, 3, 1, 2)
    o1 = o1.reshape(B, H, W, C).transpose(0, 3, 1, 2)
    return o0, o1


def kernel(x0, x1, bn1_weight, bn2_weight, bn_threshold):
    return _run(x0, x1, bn1_weight, bn2_weight, bn_threshold)
